# baseline (speedup 1.0000x reference)
import functools

import jax
import jax.numpy as jnp
from jax import lax
from jax.experimental import pallas as pl
from jax.experimental.pallas import tpu as pltpu
from jax.experimental.pallas import tpu_sc as plsc

_N = 10000
_E = 160000
_D_IN = 312
_H = 128

_NC = 2
_NS = 16
_N_PAD = 10240
_E_PAD = 163840
_K = 64
_EDGES_PER_TILE = _E_PAD // (_NC * _NS)
_NBATCH = _EDGES_PER_TILE // _K
_ROWS_PER_TILE = _N_PAD // _NS
_ZROWS = 64

_RB = 1000


def _mm_body(x_ref, w_ref, o_ref):
    o_ref[...] = jnp.dot(x_ref[...], w_ref[...], preferred_element_type=jnp.float32)


def _mm(x, w):
    n, d = x.shape
    h = w.shape[1]
    return pl.pallas_call(
        _mm_body,
        grid=(n // _RB,),
        in_specs=[
            pl.BlockSpec((_RB, d), lambda i: (i, 0)),
            pl.BlockSpec((d, h), lambda i: (0, 0)),
        ],
        out_specs=pl.BlockSpec((_RB, h), lambda i: (i, 0)),
        out_shape=jax.ShapeDtypeStruct((n, h), jnp.float32),
    )(x, w)


def _mm_bias_body(x_ref, w_ref, b_ref, o_ref):
    o_ref[...] = (jnp.dot(x_ref[...], w_ref[...], preferred_element_type=jnp.float32)
                  + b_ref[...])


def _mm_bias(x, w, b):
    n, d = x.shape
    h = w.shape[1]
    return pl.pallas_call(
        _mm_bias_body,
        grid=(n // _RB,),
        in_specs=[
            pl.BlockSpec((_RB, d), lambda i: (i, 0)),
            pl.BlockSpec((d, h), lambda i: (0, 0)),
            pl.BlockSpec((1, h), lambda i: (0, 0)),
        ],
        out_specs=pl.BlockSpec((_RB, h), lambda i: (i, 0)),
        out_shape=jax.ShapeDtypeStruct((n, h), jnp.float32),
    )(x, w, b.reshape(1, h))


def _mid_body(agg_ref, cnt_ref, pre1_ref, wl_ref, wr_ref, b_ref, hl_ref, pre2_ref):
    a = agg_ref[0] + agg_ref[1]
    cnt = cnt_ref[0] + cnt_ref[1]
    inv = 1.0 / jnp.maximum(cnt, 1.0)
    h = jnp.maximum(a * inv + pre1_ref[...], 0.0)
    hl_ref[...] = jnp.dot(h, wl_ref[...], preferred_element_type=jnp.float32)
    pre2_ref[...] = jnp.dot(h, wr_ref[...], preferred_element_type=jnp.float32) + b_ref[...]


def _mid(agg, cnt3, pre1, wl, wr, b):
    h = wl.shape[1]
    return pl.pallas_call(
        _mid_body,
        grid=(_N // _RB,),
        in_specs=[
            pl.BlockSpec((_NC, _RB, _H), lambda i: (0, i, 0)),
            pl.BlockSpec((_NC, _RB, 1), lambda i: (0, i, 0)),
            pl.BlockSpec((_RB, _H), lambda i: (i, 0)),
            pl.BlockSpec((_H, h), lambda i: (0, 0)),
            pl.BlockSpec((_H, h), lambda i: (0, 0)),
            pl.BlockSpec((1, h), lambda i: (0, 0)),
        ],
        out_specs=[
            pl.BlockSpec((_RB, h), lambda i: (i, 0)),
            pl.BlockSpec((_RB, h), lambda i: (i, 0)),
        ],
        out_shape=[
            jax.ShapeDtypeStruct((_N, h), jnp.float32),
            jax.ShapeDtypeStruct((_N, h), jnp.float32),
        ],
    )(agg, cnt3, pre1, wl, wr, b.reshape(1, h))


def _final_body(agg_ref, cnt_ref, pre2_ref, out_ref):
    a = agg_ref[0] + agg_ref[1]
    cnt = cnt_ref[0] + cnt_ref[1]
    inv = 1.0 / jnp.maximum(cnt, 1.0)
    out_ref[...] = a * inv + pre2_ref[...]


def _final(agg, cnt3, pre2):
    return pl.pallas_call(
        _final_body,
        grid=(_N // _RB,),
        in_specs=[
            pl.BlockSpec((_NC, _RB, _H), lambda i: (0, i, 0)),
            pl.BlockSpec((_NC, _RB, 1), lambda i: (0, i, 0)),
            pl.BlockSpec((_RB, _H), lambda i: (i, 0)),
        ],
        out_specs=pl.BlockSpec((_RB, _H), lambda i: (i, 0)),
        out_shape=jax.ShapeDtypeStruct((_N, _H), jnp.float32),
    )(agg, cnt3, pre2)


_NBUF = 4
_D = 2


def _make_seg_sum(with_cnt):
    mesh = plsc.VectorSubcoreMesh(core_axis_name="c", subcore_axis_name="s")
    out_type = [jax.ShapeDtypeStruct((_NC, _N_PAD, _H), jnp.float32)]
    scratch = [
        pltpu.VMEM_SHARED((_N_PAD, _H), jnp.float32),
        pltpu.VMEM((_NBATCH, _K), jnp.int32),
        pltpu.VMEM((_NBUF, _K), jnp.int32),
        pltpu.VMEM((_NBUF, _K), jnp.int32),
        pltpu.VMEM((_NBUF, _K, _H), jnp.float32),
    ] + [pltpu.SemaphoreType.DMA for _ in range(2 * _NBUF + 1)]
    if with_cnt:
        out_type.append(jax.ShapeDtypeStruct((_NC, _N_PAD), jnp.float32))
        scratch += [
            pltpu.VMEM_SHARED((_N_PAD,), jnp.float32),
            pltpu.VMEM((_K,), jnp.float32),
            pltpu.VMEM((_ROWS_PER_TILE,), jnp.float32),
        ]

    def body(xl, comb, *rest):
        if with_cnt:
            (out, cnt_out, accum, combb, srcr, dstr, rows, *more) = rest
            sems = more[:2 * _NBUF + 1]
            cnt_acc, ones_v, stage1 = more[2 * _NBUF + 1:]
        else:
            (out, accum, combb, srcr, dstr, rows, *sems) = rest
        gsems = sems[:_NBUF]
        ssems = sems[_NBUF:2 * _NBUF]
        csem = sems[2 * _NBUF]
        c = lax.axis_index("c")
        s = lax.axis_index("s")
        tid = c * _NS + s
        rbase = s * _ROWS_PER_TILE

        pltpu.async_copy(comb.at[tid], combb, gsems[0])

        def decode(b, j):
            for q in range(_K // 16):
                cv = combb[b, pl.ds(q * 16, 16)]
                srcr[j, pl.ds(q * 16, 16)] = jnp.bitwise_and(cv, 16383)
                dstr[j, pl.ds(q * 16, 16)] = lax.shift_right_logical(cv, 14)

        zv = jnp.zeros((16,), jnp.float32)

        def zrow(r, carry):
            for j in range(_H // 16):
                rows[0, r, pl.ds(j * 16, 16)] = zv
            return carry

        lax.fori_loop(0, _K, zrow, 0)

        def zcopy(k, carry):
            pltpu.async_copy(rows.at[0], accum.at[pl.ds(rbase + k * _K, _K)],
                             ssems[0])
            return carry

        lax.fori_loop(0, _ROWS_PER_TILE // _K, zcopy, 0)

        def zdrain(k, carry):
            pltpu.make_async_copy(
                rows.at[0], accum.at[pl.ds(rbase + k * _K, _K)], ssems[0]).wait()
            return carry

        if with_cnt:
            ov = jnp.ones((16,), jnp.float32)
            for j in range(_K // 16):
                ones_v[pl.ds(j * 16, 16)] = ov

            def z1(r, carry):
                stage1[pl.ds(r * 16, 16)] = zv
                return carry

            lax.fori_loop(0, _ROWS_PER_TILE // 16, z1, 0)
            pltpu.sync_copy(stage1, cnt_acc.at[pl.ds(rbase, _ROWS_PER_TILE)])

        lax.fori_loop(0, _ROWS_PER_TILE // _K, zdrain, 0)
        pltpu.make_async_copy(comb.at[tid], combb, gsems[0]).wait()
        plsc.subcore_barrier()

        def gstart(i):
            pltpu.async_copy(xl.at[srcr.at[i]], rows.at[i], gsems[i])

        def gwait(i):
            pltpu.make_async_copy(xl.at[srcr.at[i]], rows.at[i], gsems[i]).wait()

        def sstart(i):
            pltpu.async_copy(rows.at[i], accum.at[dstr.at[i]], ssems[i], add=True)

        def swait(i):
            pltpu.make_async_copy(rows.at[i], accum.at[dstr.at[i]], ssems[i]).wait()

        def cstart(i):
            pltpu.async_copy(ones_v, cnt_acc.at[dstr.at[i]], csem, add=True)

        def cwait(i):
            pltpu.make_async_copy(ones_v, cnt_acc.at[dstr.at[i]], csem).wait()

        def visit(b, i, j, do_swait, do_gstart):
            gwait(i)
            sstart(i)
            if with_cnt:
                cstart(i)
            if do_swait:
                swait(j)
                if with_cnt:
                    cwait(j)
            if do_gstart:
                decode(b + _D, j)
                gstart(j)

        for b in range(_D):
            decode(b, b % _NBUF)
            gstart(b % _NBUF)
        for b in range(_NBUF):
            visit(b, b % _NBUF, (b + _D) % _NBUF,
                  b >= _NBUF - _D, b < _NBATCH - _D)

        def blk(g, carry):
            for k in range(_NBUF):
                b = g * _NBUF + k
                visit(b, k, (k + _D) % _NBUF, True, True)
            return carry

        lax.fori_loop(1, _NBATCH // _NBUF - 1, blk, 0)
        for k in range(_NBUF):
            b = _NBATCH - _NBUF + k
            visit(b, k, (k + _D) % _NBUF, True, k < _NBUF - _D)
        for b in range(_NBATCH - _D, _NBATCH):
            swait(b % _NBUF)
            if with_cnt:
                cwait(b % _NBUF)

        plsc.subcore_barrier()

        pltpu.sync_copy(accum.at[pl.ds(rbase, _ROWS_PER_TILE)],
                        out.at[c, pl.ds(rbase, _ROWS_PER_TILE)])
        if with_cnt:
            pltpu.sync_copy(cnt_acc.at[pl.ds(rbase, _ROWS_PER_TILE)],
                            cnt_out.at[c, pl.ds(rbase, _ROWS_PER_TILE)])

    return functools.partial(
        pl.kernel, mesh=mesh, out_type=out_type, scratch_types=scratch
    )(body)


_seg_sum_cnt = _make_seg_sum(True)
_seg_sum = _make_seg_sum(False)


def kernel(x, edge_index, W1_l, b1, W1_r, W2_l, b2, W2_r):
    src = edge_index[0]
    dst = edge_index[1]
    pad = _E_PAD - _E
    ar = jnp.arange(pad, dtype=jnp.int32)
    src_p = jnp.concatenate([src, ar % _N])
    dst_p = jnp.concatenate([dst, _N + ar % (_N_PAD - _N)])
    comb = (src_p + dst_p * 16384).reshape(_NC * _NS, _NBATCH, _K)

    xl = _mm(x, W1_l)
    agg1, cnt = _seg_sum_cnt(xl, comb)
    pre1 = _mm_bias(x, W1_r, b1)
    cnt3 = cnt.reshape(_NC, _N_PAD, 1)
    hl, pre2 = _mid(agg1, cnt3, pre1, W2_l, W2_r, b2)
    (agg2,) = _seg_sum(hl, comb)
    return _final(agg2, cnt3, pre2)


# speedup vs baseline: 1.0009x; 1.0009x over previous
import functools

import jax
import jax.numpy as jnp
from jax import lax
from jax.experimental import pallas as pl
from jax.experimental.pallas import tpu as pltpu
from jax.experimental.pallas import tpu_sc as plsc

_N = 10000
_E = 160000
_D_IN = 312
_H = 128

_NC = 2
_NS = 16
_N_PAD = 10240
_E_PAD = 163840
_K = 64
_EDGES_PER_TILE = _E_PAD // (_NC * _NS)
_NBATCH = _EDGES_PER_TILE // _K
_ROWS_PER_TILE = _N_PAD // _NS

_RB = 1000


def _mm_body(x_ref, w_ref, o_ref):
    o_ref[...] = jnp.dot(x_ref[...], w_ref[...], preferred_element_type=jnp.float32)


def _mm(x, w):
    n, d = x.shape
    h = w.shape[1]
    return pl.pallas_call(
        _mm_body,
        grid=(n // _RB,),
        in_specs=[
            pl.BlockSpec((_RB, d), lambda i: (i, 0)),
            pl.BlockSpec((d, h), lambda i: (0, 0)),
        ],
        out_specs=pl.BlockSpec((_RB, h), lambda i: (i, 0)),
        out_shape=jax.ShapeDtypeStruct((n, h), jnp.float32),
    )(x, w)


def _mm_bias_body(x_ref, w_ref, b_ref, o_ref):
    o_ref[...] = (jnp.dot(x_ref[...], w_ref[...], preferred_element_type=jnp.float32)
                  + b_ref[...])


def _mm_bias(x, w, b):
    n, d = x.shape
    h = w.shape[1]
    return pl.pallas_call(
        _mm_bias_body,
        grid=(n // _RB,),
        in_specs=[
            pl.BlockSpec((_RB, d), lambda i: (i, 0)),
            pl.BlockSpec((d, h), lambda i: (0, 0)),
            pl.BlockSpec((1, h), lambda i: (0, 0)),
        ],
        out_specs=pl.BlockSpec((_RB, h), lambda i: (i, 0)),
        out_shape=jax.ShapeDtypeStruct((n, h), jnp.float32),
    )(x, w, b.reshape(1, h))


def _mid_body(agg_ref, cnt_ref, pre1_ref, wl_ref, wr_ref, b_ref, hl_ref, pre2_ref):
    a = agg_ref[0] + agg_ref[1]
    cnt = cnt_ref[0] + cnt_ref[1]
    inv = 1.0 / jnp.maximum(cnt, 1.0)
    h = jnp.maximum(a * inv + pre1_ref[...], 0.0)
    hl_ref[...] = jnp.dot(h, wl_ref[...], preferred_element_type=jnp.float32)
    pre2_ref[...] = jnp.dot(h, wr_ref[...], preferred_element_type=jnp.float32) + b_ref[...]


def _mid(agg, cnt3, pre1, wl, wr, b):
    h = wl.shape[1]
    return pl.pallas_call(
        _mid_body,
        grid=(_N // _RB,),
        in_specs=[
            pl.BlockSpec((_NC, _RB, _H), lambda i: (0, i, 0)),
            pl.BlockSpec((_NC, _RB, 1), lambda i: (0, i, 0)),
            pl.BlockSpec((_RB, _H), lambda i: (i, 0)),
            pl.BlockSpec((_H, h), lambda i: (0, 0)),
            pl.BlockSpec((_H, h), lambda i: (0, 0)),
            pl.BlockSpec((1, h), lambda i: (0, 0)),
        ],
        out_specs=[
            pl.BlockSpec((_RB, h), lambda i: (i, 0)),
            pl.BlockSpec((_RB, h), lambda i: (i, 0)),
        ],
        out_shape=[
            jax.ShapeDtypeStruct((_N, h), jnp.float32),
            jax.ShapeDtypeStruct((_N, h), jnp.float32),
        ],
    )(agg, cnt3, pre1, wl, wr, b.reshape(1, h))


def _final_body(agg_ref, cnt_ref, pre2_ref, out_ref):
    a = agg_ref[0] + agg_ref[1]
    cnt = cnt_ref[0] + cnt_ref[1]
    inv = 1.0 / jnp.maximum(cnt, 1.0)
    out_ref[...] = a * inv + pre2_ref[...]


def _final(agg, cnt3, pre2):
    return pl.pallas_call(
        _final_body,
        grid=(_N // _RB,),
        in_specs=[
            pl.BlockSpec((_NC, _RB, _H), lambda i: (0, i, 0)),
            pl.BlockSpec((_NC, _RB, 1), lambda i: (0, i, 0)),
            pl.BlockSpec((_RB, _H), lambda i: (i, 0)),
        ],
        out_specs=pl.BlockSpec((_RB, _H), lambda i: (i, 0)),
        out_shape=jax.ShapeDtypeStruct((_N, _H), jnp.float32),
    )(agg, cnt3, pre2)


_NBUF = 4
_D = 2


def _make_seg_sum(with_cnt):
    mesh = plsc.VectorSubcoreMesh(core_axis_name="c", subcore_axis_name="s")
    out_type = [jax.ShapeDtypeStruct((_NC, _N_PAD, _H), jnp.float32)]
    scratch = [
        pltpu.VMEM_SHARED((_N_PAD, _H), jnp.float32),
        pltpu.VMEM((_NBATCH, _K), jnp.int32),
        pltpu.VMEM((_NBUF, _K), jnp.int32),
        pltpu.VMEM((_NBUF, _K), jnp.int32),
        pltpu.VMEM((_NBUF, _K, _H), jnp.float32),
    ] + [pltpu.SemaphoreType.DMA for _ in range(2 * _NBUF + 1)]
    if with_cnt:
        out_type.append(jax.ShapeDtypeStruct((_NC, _N_PAD), jnp.float32))
        scratch += [
            pltpu.VMEM_SHARED((_N_PAD,), jnp.float32),
            pltpu.VMEM((_K,), jnp.float32),
            pltpu.VMEM((_ROWS_PER_TILE,), jnp.float32),
        ]

    def body(xl, comb, *rest):
        if with_cnt:
            (out, cnt_out, accum, combb, srcr, dstr, rows, *more) = rest
            sems = more[:2 * _NBUF + 1]
            cnt_acc, ones_v, stage1 = more[2 * _NBUF + 1:]
        else:
            (out, accum, combb, srcr, dstr, rows, *sems) = rest
        gsems = sems[:_NBUF]
        ssems = sems[_NBUF:2 * _NBUF]
        csem = sems[2 * _NBUF]
        c = lax.axis_index("c")
        s = lax.axis_index("s")
        tid = c * _NS + s
        rbase = s * _ROWS_PER_TILE

        pltpu.async_copy(comb.at[tid], combb, gsems[0])

        def decode(b, j):
            for q in range(_K // 16):
                cv = combb[b, pl.ds(q * 16, 16)]
                srcr[j, pl.ds(q * 16, 16)] = jnp.bitwise_and(cv, 16383)
                dstr[j, pl.ds(q * 16, 16)] = lax.shift_right_logical(cv, 14)

        zv = jnp.zeros((16,), jnp.float32)

        def zrow(r, carry):
            for j in range(_H // 16):
                rows[0, r, pl.ds(j * 16, 16)] = zv
            return carry

        lax.fori_loop(0, _K, zrow, 0)

        def zcopy(k, carry):
            pltpu.async_copy(rows.at[0], accum.at[pl.ds(rbase + k * _K, _K)],
                             ssems[0])
            return carry

        lax.fori_loop(0, _ROWS_PER_TILE // _K, zcopy, 0)

        def zdrain(k, carry):
            pltpu.make_async_copy(
                rows.at[0], accum.at[pl.ds(rbase + k * _K, _K)], ssems[0]).wait()
            return carry

        if with_cnt:
            ov = jnp.ones((16,), jnp.float32)
            for j in range(_K // 16):
                ones_v[pl.ds(j * 16, 16)] = ov

            def z1(r, carry):
                stage1[pl.ds(r * 16, 16)] = zv
                return carry

            lax.fori_loop(0, _ROWS_PER_TILE // 16, z1, 0)
            pltpu.sync_copy(stage1, cnt_acc.at[pl.ds(rbase, _ROWS_PER_TILE)])

        lax.fori_loop(0, _ROWS_PER_TILE // _K, zdrain, 0)
        pltpu.make_async_copy(comb.at[tid], combb, gsems[0]).wait()
        plsc.subcore_barrier()

        def gstart(i):
            pltpu.async_copy(xl.at[srcr.at[i]], rows.at[i], gsems[i])

        def gwait(i):
            pltpu.make_async_copy(xl.at[srcr.at[i]], rows.at[i], gsems[i]).wait()

        def sstart(i):
            pltpu.async_copy(rows.at[i], accum.at[dstr.at[i]], ssems[i], add=True)

        def swait(i):
            pltpu.make_async_copy(rows.at[i], accum.at[dstr.at[i]], ssems[i]).wait()

        def cstart(i):
            pltpu.async_copy(ones_v, cnt_acc.at[dstr.at[i]], csem, add=True)

        def cwait(i):
            pltpu.make_async_copy(ones_v, cnt_acc.at[dstr.at[i]], csem).wait()

        def visit(b, i, j, do_swait, do_gstart):
            gwait(i)
            sstart(i)
            if with_cnt:
                cstart(i)
            if do_swait:
                swait(j)
                if with_cnt:
                    cwait(j)
            if do_gstart:
                decode(b + _D, j)
                gstart(j)

        for b in range(_D):
            decode(b, b % _NBUF)
            gstart(b % _NBUF)
        for b in range(_NBUF):
            visit(b, b % _NBUF, (b + _D) % _NBUF,
                  b >= _NBUF - _D, b < _NBATCH - _D)

        def blk(g, carry):
            for k in range(_NBUF):
                b = g * _NBUF + k
                visit(b, k, (k + _D) % _NBUF, True, True)
            return carry

        lax.fori_loop(1, _NBATCH // _NBUF - 1, blk, 0)
        for k in range(_NBUF):
            b = _NBATCH - _NBUF + k
            visit(b, k, (k + _D) % _NBUF, True, k < _NBUF - _D)
        for b in range(_NBATCH - _D, _NBATCH):
            swait(b % _NBUF)
            if with_cnt:
                cwait(b % _NBUF)

        plsc.subcore_barrier()

        pltpu.sync_copy(accum.at[pl.ds(rbase, _ROWS_PER_TILE)],
                        out.at[c, pl.ds(rbase, _ROWS_PER_TILE)])
        if with_cnt:
            pltpu.sync_copy(cnt_acc.at[pl.ds(rbase, _ROWS_PER_TILE)],
                            cnt_out.at[c, pl.ds(rbase, _ROWS_PER_TILE)])

    return functools.partial(
        pl.kernel, mesh=mesh, out_type=out_type, scratch_types=scratch
    )(body)


_seg_sum_cnt = _make_seg_sum(True)
_seg_sum = _make_seg_sum(False)


def kernel(x, edge_index, W1_l, b1, W1_r, W2_l, b2, W2_r):
    src = edge_index[0]
    dst = edge_index[1]
    pad = _E_PAD - _E
    ar = jnp.arange(pad, dtype=jnp.int32)
    src_p = jnp.concatenate([src, ar % _N])
    dst_p = jnp.concatenate([dst, _N + ar % (_N_PAD - _N)])
    comb = (src_p + dst_p * 16384).reshape(_NC * _NS, _NBATCH, _K)

    xl = _mm(x, W1_l)
    agg1, cnt = _seg_sum_cnt(xl, comb)
    pre1 = _mm_bias(x, W1_r, b1)
    cnt3 = cnt.reshape(_NC, _N_PAD, 1)
    hl, pre2 = _mid(agg1, cnt3, pre1, W2_l, W2_r, b2)
    (agg2,) = _seg_sum(hl, comb)
    return _final(agg2, cnt3, pre2)
